# baseline (speedup 1.0000x reference)
import functools

import jax
import jax.numpy as jnp
from jax import lax
from jax.experimental import pallas as pl
from jax.experimental.pallas import tpu as pltpu
from jax.experimental.pallas import tpu_sc as plsc

B = 16384
D_IN = 256
K = 64
NL = 8192
NC = 2
NS = 16
NW = NC * NS
BPW = B // NW
NCHUNK = 4


def _mm_body(x_ref, w_ref, o_ref):
    o_ref[...] = jnp.dot(x_ref[...], w_ref[...],
                         preferred_element_type=jnp.float32)


_matmul = pl.pallas_call(
    _mm_body,
    grid=(8,),
    in_specs=[
        pl.BlockSpec((B // 8, D_IN), lambda i: (i, 0)),
        pl.BlockSpec((D_IN, K), lambda i: (0, 0)),
    ],
    out_specs=pl.BlockSpec((B // 8, K), lambda i: (i, 0)),
    out_shape=jax.ShapeDtypeStruct((B, K), jnp.float32),
)


_sc_mesh = plsc.VectorSubcoreMesh(core_axis_name="c", subcore_axis_name="s")


@functools.partial(
    pl.kernel,
    out_type=[
        jax.ShapeDtypeStruct((NC * NL, K), jnp.float32),
        jax.ShapeDtypeStruct((B, 16), jnp.float32),
    ],
    mesh=_sc_mesh,
    scratch_types=[
        pltpu.VMEM((BPW, K), jnp.float32),
        pltpu.VMEM((BPW, K), jnp.float32),
        pltpu.VMEM((NCHUNK, 128), jnp.int32),
        pltpu.VMEM((BPW, 16), jnp.float32),
        pltpu.VMEM((64, K), jnp.float32),
        pltpu.VMEM_SHARED((NL, K), jnp.float32),
        pltpu.SemaphoreType.DMA,
        pltpu.SemaphoreType.DMA,
        pltpu.SemaphoreType.DMA,
        pltpu.SemaphoreType.DMA,
        pltpu.SemaphoreType.DMA,
    ],
    compiler_params=pltpu.CompilerParams(use_tc_tiling_on_sc=False),
)
def _sc_update(emb_hbm, lab_hbm, grid_hbm, dparts_hbm, ss_hbm,
               emb_v, rows_v, idx_v, ss_v, zeros_v, delta_sp,
               sem_e, sem_i, sem_g, sem_s, sem_z):
    cid = lax.axis_index("c")
    sid = lax.axis_index("s")
    wid = cid * NS + sid
    base = pl.multiple_of(wid * BPW, BPW)

    cp_e = pltpu.async_copy(emb_hbm.at[pl.ds(base, BPW)], emb_v, sem_e)
    cp_i = pltpu.async_copy(
        lab_hbm.at[pl.ds(pl.multiple_of(wid * NCHUNK, NCHUNK), NCHUNK)],
        idx_v, sem_i)

    zero16 = jnp.zeros((16,), jnp.float32)

    def _zero_body(i, carry):
        for j in range(K // 16):
            zeros_v[i, pl.ds(16 * j, 16)] = zero16
        return carry

    lax.fori_loop(0, 64, _zero_body, 0)
    zcps = []
    for t in range(BPW // 64):
        off = pl.multiple_of(sid * BPW + t * 64, 64)
        zcps.append(pltpu.async_copy(zeros_v, delta_sp.at[pl.ds(off, 64)],
                                     sem_z))

    cp_i.wait()
    one16 = jnp.full((16,), 1, jnp.int32)
    for j in range(NCHUNK):
        for q in range(128 // 16):
            sl = pl.ds(16 * q, 16)
            idx_v[j, sl] = idx_v[j, sl] - one16

    gcps = [
        pltpu.async_copy(grid_hbm.at[idx_v.at[j]],
                         rows_v.at[pl.ds(128 * j, 128)], sem_g)
        for j in range(NCHUNK)
    ]
    for cp in zcps:
        cp.wait()
    plsc.subcore_barrier()
    cp_e.wait()

    scps = []
    for j in range(NCHUNK):
        gcps[j].wait()

        def _sub_body(i, carry, j=j):
            for m in range(4):
                r = 4 * i + 128 * j + m
                acc = zero16
                for q in range(K // 16):
                    sl = pl.ds(16 * q, 16)
                    d = emb_v[r, sl] - rows_v[r, sl]
                    rows_v[r, sl] = d
                    acc = acc + d * d
                ss_v[r] = acc
            return carry

        lax.fori_loop(0, 32, _sub_body, 0)
        scps.append(pltpu.async_copy(rows_v.at[pl.ds(128 * j, 128)],
                                     delta_sp.at[idx_v.at[j]], sem_s,
                                     add=True))

    cp_ss = pltpu.async_copy(ss_v, ss_hbm.at[pl.ds(base, BPW)], sem_e)
    for cp in scps:
        cp.wait()
    plsc.subcore_barrier()

    src_off = pl.multiple_of(sid * BPW, BPW)
    dst_off = pl.multiple_of(cid * NL + sid * BPW, BPW)
    pltpu.sync_copy(delta_sp.at[pl.ds(src_off, BPW)],
                    dparts_hbm.at[pl.ds(dst_off, BPW)])
    cp_ss.wait()


def _fin_body(grid_ref, dp_ref, ss_ref, out_ref, loss_ref):
    out_ref[...] = grid_ref[...] + 1e-3 * (dp_ref[0] + dp_ref[1])
    d = jnp.sqrt(jnp.sum(ss_ref[...], axis=1))
    loss_ref[0, 0] = jnp.sum(jnp.maximum(d - 0.2, 0.0)) * (1.0 / B)


_finish = pl.pallas_call(
    _fin_body,
    out_shape=(
        jax.ShapeDtypeStruct((NL, K), jnp.float32),
        jax.ShapeDtypeStruct((1, 1), jnp.float32),
    ),
    out_specs=(
        pl.BlockSpec(memory_space=pltpu.VMEM),
        pl.BlockSpec(memory_space=pltpu.SMEM),
    ),
)


def kernel(images, labels, W, grid):
    emb = _matmul(images, W)
    lab2d = labels.reshape(NW * NCHUNK, 128)
    dparts, ss = _sc_update(emb, lab2d, grid)
    new_grid, loss = _finish(grid, dparts.reshape(NC, NL, K), ss)
    return loss.reshape(()), new_grid


# speedup vs baseline: 1.1498x; 1.1498x over previous
import functools

import jax
import jax.numpy as jnp
from jax import lax
from jax.experimental import pallas as pl
from jax.experimental.pallas import tpu as pltpu
from jax.experimental.pallas import tpu_sc as plsc

B = 16384
D_IN = 256
K = 64
NL = 8192
NC = 2
NS = 16
NW = NC * NS
BPW = B // NW
NCHUNK = 4


def _mm_body(x_ref, w_ref, o_ref):
    o_ref[...] = jnp.dot(x_ref[...], w_ref[...],
                         preferred_element_type=jnp.float32)


_matmul = pl.pallas_call(
    _mm_body,
    grid=(8,),
    in_specs=[
        pl.BlockSpec((B // 8, D_IN), lambda i: (i, 0)),
        pl.BlockSpec((D_IN, K), lambda i: (0, 0)),
    ],
    out_specs=pl.BlockSpec((B // 8, K), lambda i: (i, 0)),
    out_shape=jax.ShapeDtypeStruct((B, K), jnp.float32),
)


_sc_mesh = plsc.VectorSubcoreMesh(core_axis_name="c", subcore_axis_name="s")


@functools.partial(
    pl.kernel,
    out_type=[
        jax.ShapeDtypeStruct((NC * NL, K), jnp.float32),
        jax.ShapeDtypeStruct((B, 16), jnp.float32),
    ],
    mesh=_sc_mesh,
    scratch_types=[
        pltpu.VMEM((BPW, K), jnp.float32),
        pltpu.VMEM((BPW, K), jnp.float32),
        pltpu.VMEM((NCHUNK, 128), jnp.int32),
        pltpu.VMEM((BPW, 16), jnp.float32),
        pltpu.VMEM((64, K), jnp.float32),
        pltpu.VMEM_SHARED((NL, K), jnp.float32),
        pltpu.SemaphoreType.DMA,
        pltpu.SemaphoreType.DMA,
        pltpu.SemaphoreType.DMA,
        pltpu.SemaphoreType.DMA,
        pltpu.SemaphoreType.DMA,
    ],
    compiler_params=pltpu.CompilerParams(use_tc_tiling_on_sc=False),
)
def _sc_update(emb_hbm, lab_hbm, grid_hbm, dparts_hbm, ss_hbm,
               emb_v, rows_v, idx_v, ss_v, zeros_v, delta_sp,
               sem_e, sem_i, sem_g, sem_s, sem_z):
    cid = lax.axis_index("c")
    sid = lax.axis_index("s")
    wid = cid * NS + sid
    base = pl.multiple_of(wid * BPW, BPW)

    cp_e = pltpu.async_copy(emb_hbm.at[pl.ds(base, BPW)], emb_v, sem_e)
    cp_i = pltpu.async_copy(
        lab_hbm.at[pl.ds(pl.multiple_of(wid * NCHUNK, NCHUNK), NCHUNK)],
        idx_v, sem_i)

    zero16 = jnp.zeros((16,), jnp.float32)

    def _zero_body(i, carry):
        for j in range(K // 16):
            zeros_v[i, pl.ds(16 * j, 16)] = zero16
        return carry

    lax.fori_loop(0, 64, _zero_body, 0)
    zcps = []
    for t in range(BPW // 64):
        off = pl.multiple_of(sid * BPW + t * 64, 64)
        zcps.append(pltpu.async_copy(zeros_v, delta_sp.at[pl.ds(off, 64)],
                                     sem_z))

    cp_i.wait()
    one16 = jnp.full((16,), 1, jnp.int32)
    for j in range(NCHUNK):
        for q in range(128 // 16):
            sl = pl.ds(16 * q, 16)
            idx_v[j, sl] = idx_v[j, sl] - one16

    gcps = [
        pltpu.async_copy(grid_hbm.at[idx_v.at[j]],
                         rows_v.at[pl.ds(128 * j, 128)], sem_g)
        for j in range(NCHUNK)
    ]
    for cp in zcps:
        cp.wait()
    plsc.subcore_barrier()
    cp_e.wait()

    scps = []
    for j in range(NCHUNK):
        gcps[j].wait()

        def _sub_body(i, carry, j=j):
            for m in range(4):
                r = 4 * i + 128 * j + m
                acc = zero16
                for q in range(K // 16):
                    sl = pl.ds(16 * q, 16)
                    d = emb_v[r, sl] - rows_v[r, sl]
                    rows_v[r, sl] = d
                    acc = acc + d * d
                ss_v[r] = acc
            return carry

        lax.fori_loop(0, 32, _sub_body, 0)
        scps.append(pltpu.async_copy(rows_v.at[pl.ds(128 * j, 128)],
                                     delta_sp.at[idx_v.at[j]], sem_s,
                                     add=True))

    cp_ss = pltpu.async_copy(ss_v, ss_hbm.at[pl.ds(base, BPW)], sem_e)
    for cp in scps:
        cp.wait()
    plsc.subcore_barrier()

    src_off = pl.multiple_of(sid * BPW, BPW)
    dst_off = pl.multiple_of(cid * NL + sid * BPW, BPW)
    pltpu.sync_copy(delta_sp.at[pl.ds(src_off, BPW)],
                    dparts_hbm.at[pl.ds(dst_off, BPW)])
    cp_ss.wait()


_FSTEPS = 4


def _fin_body(grid_ref, d0_ref, d1_ref, ss_ref, sel_ref, out_ref, loss_ref):
    i = pl.program_id(0)
    out_ref[...] = grid_ref[...] + 1e-3 * (d0_ref[0] + d1_ref[0])
    d2 = jnp.dot(ss_ref[...], sel_ref[...],
                 preferred_element_type=jnp.float32)
    d = jnp.sqrt(d2)
    part = jnp.sum(jnp.maximum(d - 0.2, 0.0)) * (1.0 / B)

    @pl.when(i == 0)
    def _():
        loss_ref[0, 0] = part

    @pl.when(i != 0)
    def _():
        loss_ref[0, 0] += part


_finish = pl.pallas_call(
    _fin_body,
    grid=(_FSTEPS,),
    in_specs=[
        pl.BlockSpec((NL // _FSTEPS, K), lambda i: (i, 0)),
        pl.BlockSpec((1, NL // _FSTEPS, K), lambda i: (0, i, 0)),
        pl.BlockSpec((1, NL // _FSTEPS, K), lambda i: (1, i, 0)),
        pl.BlockSpec((B // 8 // _FSTEPS, 128), lambda i: (i, 0)),
        pl.BlockSpec((128, 8), lambda i: (0, 0)),
    ],
    out_specs=(
        pl.BlockSpec((NL // _FSTEPS, K), lambda i: (i, 0)),
        pl.BlockSpec(memory_space=pltpu.SMEM),
    ),
    out_shape=(
        jax.ShapeDtypeStruct((NL, K), jnp.float32),
        jax.ShapeDtypeStruct((1, 1), jnp.float32),
    ),
)


def kernel(images, labels, W, grid):
    emb = _matmul(images, W)
    lab2d = labels.reshape(NW * NCHUNK, 128)
    dparts, ss = _sc_update(emb, lab2d, grid)
    dp3 = dparts.reshape(NC, NL, K)
    sel = (jnp.arange(128, dtype=jnp.int32)[:, None] // 16
           == jnp.arange(8, dtype=jnp.int32)[None, :]).astype(jnp.float32)
    new_grid, loss = _finish(grid, dp3, dp3, ss.reshape(B // 8, 128), sel)
    return loss.reshape(()), new_grid


# speedup vs baseline: 1.3002x; 1.1308x over previous
import functools

import jax
import jax.numpy as jnp
from jax import lax
from jax.experimental import pallas as pl
from jax.experimental.pallas import tpu as pltpu
from jax.experimental.pallas import tpu_sc as plsc

B = 16384
D_IN = 256
K = 64
NL = 8192
NC = 2
NS = 16
NW = NC * NS
BPW = B // NW
NCHUNK = 4


def _mm_body(x_ref, w_ref, o_ref):
    o_ref[...] = jnp.dot(x_ref[...], w_ref[...],
                         preferred_element_type=jnp.float32)


_matmul = pl.pallas_call(
    _mm_body,
    grid=(8,),
    in_specs=[
        pl.BlockSpec((B // 8, D_IN), lambda i: (i, 0)),
        pl.BlockSpec((D_IN, K), lambda i: (0, 0)),
    ],
    out_specs=pl.BlockSpec((B // 8, K), lambda i: (i, 0)),
    out_shape=jax.ShapeDtypeStruct((B, K), jnp.float32),
)


_sc_mesh = plsc.VectorSubcoreMesh(core_axis_name="c", subcore_axis_name="s")


@functools.partial(
    pl.kernel,
    out_type=[
        jax.ShapeDtypeStruct((NC * NL, K), jnp.float32),
        jax.ShapeDtypeStruct((B, 16), jnp.float32),
    ],
    mesh=_sc_mesh,
    scratch_types=[
        pltpu.VMEM((BPW, K), jnp.float32),
        pltpu.VMEM((BPW, K), jnp.float32),
        pltpu.VMEM((NCHUNK, 128), jnp.int32),
        pltpu.VMEM((BPW, 16), jnp.float32),
        pltpu.VMEM((64, K), jnp.float32),
        pltpu.VMEM_SHARED((NL, K), jnp.float32),
        pltpu.SemaphoreType.DMA,
        pltpu.SemaphoreType.DMA,
        pltpu.SemaphoreType.DMA,
        pltpu.SemaphoreType.DMA,
        pltpu.SemaphoreType.DMA,
    ],
    compiler_params=pltpu.CompilerParams(use_tc_tiling_on_sc=False),
)
def _sc_update(emb_hbm, lab_hbm, grid_hbm, dparts_hbm, ss_hbm,
               emb_v, rows_v, idx_v, ss_v, zeros_v, delta_sp,
               sem_e, sem_i, sem_g, sem_s, sem_z):
    cid = lax.axis_index("c")
    sid = lax.axis_index("s")
    wid = cid * NS + sid
    base = pl.multiple_of(wid * BPW, BPW)

    cp_e = pltpu.async_copy(emb_hbm.at[pl.ds(base, BPW)], emb_v, sem_e)
    cp_i = pltpu.async_copy(
        lab_hbm.at[pl.ds(pl.multiple_of(wid * NCHUNK, NCHUNK), NCHUNK)],
        idx_v, sem_i)

    zero16 = jnp.zeros((16,), jnp.float32)

    def _zero_body(i, carry):
        for j in range(K // 16):
            zeros_v[i, pl.ds(16 * j, 16)] = zero16
        return carry

    lax.fori_loop(0, 64, _zero_body, 0)
    zcps = []
    for t in range(BPW // 64):
        off = pl.multiple_of(sid * BPW + t * 64, 64)
        zcps.append(pltpu.async_copy(zeros_v, delta_sp.at[pl.ds(off, 64)],
                                     sem_z))

    cp_i.wait()
    one16 = jnp.full((16,), 1, jnp.int32)
    for j in range(NCHUNK):
        for q in range(128 // 16):
            sl = pl.ds(16 * q, 16)
            idx_v[j, sl] = idx_v[j, sl] - one16

    gcps = [
        pltpu.async_copy(grid_hbm.at[idx_v.at[j]],
                         rows_v.at[pl.ds(128 * j, 128)], sem_g)
        for j in range(NCHUNK)
    ]
    for cp in zcps:
        cp.wait()
    plsc.subcore_barrier()
    cp_e.wait()

    scps = []
    for j in range(NCHUNK):
        gcps[j].wait()

        def _sub_body(i, carry, j=j):
            for m in range(4):
                r = 4 * i + 128 * j + m
                acc = zero16
                for q in range(K // 16):
                    sl = pl.ds(16 * q, 16)
                    d = emb_v[r, sl] - rows_v[r, sl]
                    rows_v[r, sl] = d
                    acc = acc + d * d
                ss_v[r] = acc
            return carry

        lax.fori_loop(0, 32, _sub_body, 0)
        scps.append(pltpu.async_copy(rows_v.at[pl.ds(128 * j, 128)],
                                     delta_sp.at[idx_v.at[j]], sem_s,
                                     add=True))

    cp_ss = pltpu.async_copy(ss_v, ss_hbm.at[pl.ds(base, BPW)], sem_e)
    for cp in scps:
        cp.wait()
    plsc.subcore_barrier()

    src_off = pl.multiple_of(sid * BPW, BPW)
    dst_off = pl.multiple_of(cid * NL + sid * BPW, BPW)
    pltpu.sync_copy(delta_sp.at[pl.ds(src_off, BPW)],
                    dparts_hbm.at[pl.ds(dst_off, BPW)])
    cp_ss.wait()


_FSTEPS = 4


def _fin_body(grid_ref, d0_ref, d1_ref, ss_ref, sel_ref, out_ref, loss_ref):
    i = pl.program_id(0)
    dsum = d0_ref[...] + d1_ref[...]
    out_ref[0::2, :] = grid_ref[0::2, :] + 1e-3 * dsum[:, :K]
    out_ref[1::2, :] = grid_ref[1::2, :] + 1e-3 * dsum[:, K:]
    d2 = jnp.dot(ss_ref[...], sel_ref[...],
                 preferred_element_type=jnp.float32)
    d = jnp.sqrt(d2)
    part = jnp.sum(jnp.maximum(d - 0.2, 0.0)) * (1.0 / B)

    @pl.when(i == 0)
    def _():
        loss_ref[0, 0] = part

    @pl.when(i != 0)
    def _():
        loss_ref[0, 0] += part


_finish = pl.pallas_call(
    _fin_body,
    grid=(_FSTEPS,),
    in_specs=[
        pl.BlockSpec((NL // _FSTEPS, K), lambda i: (i, 0)),
        pl.BlockSpec((NL // _FSTEPS // 2, 128), lambda i: (i, 0)),
        pl.BlockSpec((NL // _FSTEPS // 2, 128), lambda i: (i + _FSTEPS, 0)),
        pl.BlockSpec((B // 8 // _FSTEPS, 128), lambda i: (i, 0)),
        pl.BlockSpec((128, 8), lambda i: (0, 0)),
    ],
    out_specs=(
        pl.BlockSpec((NL // _FSTEPS, K), lambda i: (i, 0)),
        pl.BlockSpec(memory_space=pltpu.SMEM),
    ),
    out_shape=(
        jax.ShapeDtypeStruct((NL, K), jnp.float32),
        jax.ShapeDtypeStruct((1, 1), jnp.float32),
    ),
)


def kernel(images, labels, W, grid):
    emb = _matmul(images, W)
    lab2d = labels.reshape(NW * NCHUNK, 128)
    dparts, ss = _sc_update(emb, lab2d, grid)
    dp2 = dparts.reshape(NC * NL * K // 128, 128)
    sel = (jnp.arange(128, dtype=jnp.int32)[:, None] // 16
           == jnp.arange(8, dtype=jnp.int32)[None, :]).astype(jnp.float32)
    new_grid, loss = _finish(grid, dp2, dp2, ss.reshape(B // 8, 128), sel)
    return loss.reshape(()), new_grid
